# Initial kernel scaffold; baseline (speedup 1.0000x reference)
import functools

import jax
import jax.numpy as jnp
from jax import lax
from jax.experimental import pallas as pl
from jax.experimental.pallas import tpu as pltpu
from jax.experimental.pallas import tpu_sc as plsc

N = 10000
D = 128
HALF = 64
NT = 16
EB = 128
RPT = N // NT
RCH = 125
MB = 1000


def _mm_body(x_ref, k_ref, o_ref):
    o_ref[...] = jnp.dot(x_ref[...], k_ref[...],
                         preferred_element_type=jnp.float32)


_matmul = pl.pallas_call(
    _mm_body,
    grid=(2, N // MB),
    in_specs=[
        pl.BlockSpec((MB, D), lambda c, i: (i, 0)),
        pl.BlockSpec((D, HALF), lambda c, i: (0, c)),
    ],
    out_specs=pl.BlockSpec((MB, HALF), lambda c, i: (c * (N // MB) + i, 0)),
    out_shape=jax.ShapeDtypeStruct((2 * N, HALF), jnp.float32),
)


@functools.lru_cache(maxsize=None)
def _make_spmm(nblk: int, apply_filt: bool):
    mesh = plsc.VectorSubcoreMesh(core_axis_name="c", subcore_axis_name="s")
    scratch = [
        pltpu.VMEM_SHARED((N, HALF), jnp.float32),
        pltpu.VMEM((nblk, EB), jnp.int32),
        pltpu.VMEM((nblk, EB), jnp.int32),
        pltpu.VMEM((nblk, EB), jnp.float32),
        pltpu.VMEM((EB, HALF), jnp.float32),
        pltpu.VMEM((RCH, HALF), jnp.float32),
        pltpu.VMEM((N,), jnp.float32),
        pltpu.SemaphoreType.DMA,
    ]

    def body(h2, colsp, rowsp, valsp, filt, out,
             acc, colsb, rowsb, valsb, gbuf, obuf, filt_v, sem):
        c = lax.axis_index("c")
        s = lax.axis_index("s")

        def zb(i, carry):
            for f in range(HALF // 16):
                obuf[i, pl.ds(f * 16, 16)] = jnp.zeros((16,), jnp.float32)
            return carry
        lax.fori_loop(0, RCH, zb, 0)

        def zc(k2, carry):
            pltpu.sync_copy(obuf, acc.at[pl.ds(s * RPT + k2 * RCH, RCH)])
            return carry
        lax.fori_loop(0, RPT // RCH, zc, 0)

        pltpu.sync_copy(colsp.at[s], colsb)
        pltpu.sync_copy(rowsp.at[s], rowsb)
        pltpu.sync_copy(valsp.at[s], valsb)
        if apply_filt:
            pltpu.sync_copy(filt, filt_v)

        off = jnp.full((16,), c * N, jnp.int32)

        def pe(j, carry):
            for f in range(EB // 16):
                sl = pl.ds(f * 16, 16)
                colsb[j, sl] = colsb[j, sl] + off
                if apply_filt:
                    fv = plsc.load_gather(filt_v, [rowsb[j, sl]])
                    valsb[j, sl] = valsb[j, sl] * fv
            return carry
        lax.fori_loop(0, nblk, pe, 0)

        plsc.subcore_barrier()

        def blk(j, carry):
            pltpu.async_copy(h2.at[colsb.at[j]], gbuf, sem).wait()

            def se(i2, c2):
                for u in range(4):
                    i = i2 * 4 + u
                    v = plsc.load_gather(
                        valsb, [jnp.full((16,), j, jnp.int32),
                                jnp.full((16,), i, jnp.int32)])
                    for f in range(HALF // 16):
                        sl = pl.ds(f * 16, 16)
                        gbuf[i, sl] = gbuf[i, sl] * v
                return c2
            lax.fori_loop(0, EB // 4, se, 0)

            pltpu.sync_copy(gbuf, acc.at[rowsb.at[j]], add=True)
            return carry
        lax.fori_loop(0, nblk, blk, 0)

        plsc.subcore_barrier()

        def wb(k2, carry):
            base = s * RPT + k2 * RCH
            pltpu.sync_copy(acc.at[pl.ds(base, RCH)], obuf)
            pltpu.sync_copy(obuf, out.at[pl.ds(c * N + base, RCH)])
            return carry
        lax.fori_loop(0, RPT // RCH, wb, 0)

    return pl.kernel(
        body,
        out_type=jax.ShapeDtypeStruct((2 * N, HALF), jnp.float32),
        mesh=mesh,
        scratch_types=scratch,
    )


def _prep_edges(rows, cols, vals):
    e = rows.shape[0]
    nblk = -(-e // (NT * EB))
    tot = NT * nblk * EB
    pad = tot - e
    rows_p = jnp.concatenate([rows, jnp.zeros((pad,), rows.dtype)])
    cols_p = jnp.concatenate([cols, jnp.zeros((pad,), cols.dtype)])
    vals_p = jnp.concatenate([vals, jnp.zeros((pad,), vals.dtype)])
    shape = (NT, nblk, EB)
    return (nblk, cols_p.reshape(shape), rows_p.reshape(shape),
            vals_p.reshape(shape))


def kernel(x, wavelet_indices, wavelet_values, inverse_wavelet_indices,
           inverse_wavelet_values, kernel, filt):
    h2 = _matmul(x, kernel)

    nblk1, colsp1, rowsp1, valsp1 = _prep_edges(
        inverse_wavelet_indices[0], inverse_wavelet_indices[1],
        inverse_wavelet_values)
    h1 = _make_spmm(nblk1, True)(h2, colsp1, rowsp1, valsp1, filt)

    nblk2, colsp2, rowsp2, valsp2 = _prep_edges(
        wavelet_indices[0], wavelet_indices[1], wavelet_values)
    o2 = _make_spmm(nblk2, False)(h1, colsp2, rowsp2, valsp2, filt)

    return jnp.concatenate([o2[:N], o2[N:]], axis=1)


# speedup vs baseline: 3.0385x; 3.0385x over previous
import functools

import jax
import jax.numpy as jnp
from jax import lax
from jax.experimental import pallas as pl
from jax.experimental.pallas import tpu as pltpu
from jax.experimental.pallas import tpu_sc as plsc

N = 10000
D = 128
NT = 16
NW = 32
EB = 128
RPT = 640
RPL = N - 15 * RPT
RCH = 40
SBLK = 16
MB = 1000


def _mm_body(x_ref, k_ref, o_ref):
    o_ref[...] = jnp.dot(x_ref[...], k_ref[...],
                         preferred_element_type=jnp.float32)


_matmul = pl.pallas_call(
    _mm_body,
    grid=(N // MB,),
    in_specs=[
        pl.BlockSpec((MB, D), lambda i: (i, 0)),
        pl.BlockSpec((D, D), lambda i: (0, 0)),
    ],
    out_specs=pl.BlockSpec((MB, D), lambda i: (i, 0)),
    out_shape=jax.ShapeDtypeStruct((N, D), jnp.float32),
)


def _comb_filt_body(p_ref, f_ref, o_ref):
    o_ref[...] = (p_ref[0] + p_ref[1]) * f_ref[...]


_combine_filt = pl.pallas_call(
    _comb_filt_body,
    grid=(N // MB,),
    in_specs=[
        pl.BlockSpec((2, MB, D), lambda i: (0, i, 0)),
        pl.BlockSpec((MB, 1), lambda i: (i, 0)),
    ],
    out_specs=pl.BlockSpec((MB, D), lambda i: (i, 0)),
    out_shape=jax.ShapeDtypeStruct((N, D), jnp.float32),
)


def _comb_body(p_ref, o_ref):
    o_ref[...] = p_ref[0] + p_ref[1]


_combine = pl.pallas_call(
    _comb_body,
    grid=(N // MB,),
    in_specs=[pl.BlockSpec((2, MB, D), lambda i: (0, i, 0))],
    out_specs=pl.BlockSpec((MB, D), lambda i: (i, 0)),
    out_shape=jax.ShapeDtypeStruct((N, D), jnp.float32),
)


@functools.lru_cache(maxsize=None)
def _make_spmm(nblk: int):
    mesh = plsc.VectorSubcoreMesh(core_axis_name="c", subcore_axis_name="s")
    scratch = [
        pltpu.VMEM_SHARED((N, D), jnp.float32),
        pltpu.VMEM((SBLK, EB), jnp.int32),
        pltpu.VMEM((SBLK, EB), jnp.int32),
        pltpu.VMEM((SBLK, EB), jnp.float32),
        pltpu.VMEM((EB, D), jnp.float32),
        pltpu.VMEM((RCH, D), jnp.float32),
        pltpu.SemaphoreType.DMA,
    ]

    def body(h, colsp, rowsp, valsp, out,
             acc, colsb, rowsb, valsb, gbuf, obuf, sem):
        c = lax.axis_index("c")
        s = lax.axis_index("s")
        base = s * RPT
        nrow = jnp.where(s == NT - 1, RPL, RPT)

        def zb(i, carry):
            for f in range(D // 16):
                obuf[i, pl.ds(f * 16, 16)] = jnp.zeros((16,), jnp.float32)
            return carry
        lax.fori_loop(0, RCH, zb, 0)

        def zc(k2, carry):
            pltpu.sync_copy(obuf, acc.at[pl.ds(base + k2 * RCH, RCH)])
            return carry
        lax.fori_loop(0, nrow // RCH, zc, 0)

        w = s * 2 + c
        plsc.subcore_barrier()

        def sb_loop(sb, carry):
            pltpu.sync_copy(colsp.at[w].at[pl.ds(sb * SBLK, SBLK)], colsb)
            pltpu.sync_copy(rowsp.at[w].at[pl.ds(sb * SBLK, SBLK)], rowsb)
            pltpu.sync_copy(valsp.at[w].at[pl.ds(sb * SBLK, SBLK)], valsb)

            def blk(j, c1):
                pltpu.async_copy(h.at[colsb.at[j]], gbuf, sem).wait()

                def se(g, c2):
                    vv = valsb[j, pl.ds(g * 16, 16)]
                    for u in range(16):
                        i = g * 16 + u
                        v = jnp.full((16,), vv[u], jnp.float32)
                        for f in range(D // 16):
                            sl = pl.ds(f * 16, 16)
                            gbuf[i, sl] = gbuf[i, sl] * v
                    return c2
                lax.fori_loop(0, EB // 16, se, 0)

                pltpu.sync_copy(gbuf, acc.at[rowsb.at[j]], add=True)
                return c1
            lax.fori_loop(0, SBLK, blk, 0)
            return carry
        lax.fori_loop(0, nblk // SBLK, sb_loop, 0)

        plsc.subcore_barrier()

        def wb(k2, carry):
            pltpu.sync_copy(acc.at[pl.ds(base + k2 * RCH, RCH)], obuf)
            pltpu.sync_copy(obuf, out.at[c].at[pl.ds(base + k2 * RCH, RCH)])
            return carry
        lax.fori_loop(0, nrow // RCH, wb, 0)

    return pl.kernel(
        body,
        out_type=jax.ShapeDtypeStruct((2, N, D), jnp.float32),
        mesh=mesh,
        scratch_types=scratch,
    )


def _prep_edges(rows, cols, vals):
    e = rows.shape[0]
    nblk = -(-e // (NW * EB * SBLK)) * SBLK
    tot = NW * nblk * EB
    pad = tot - e
    rows_p = jnp.concatenate([rows, jnp.zeros((pad,), rows.dtype)])
    cols_p = jnp.concatenate([cols, jnp.zeros((pad,), cols.dtype)])
    vals_p = jnp.concatenate([vals, jnp.zeros((pad,), vals.dtype)])
    shape = (NW, nblk, EB)
    return (nblk, cols_p.reshape(shape), rows_p.reshape(shape),
            vals_p.reshape(shape))


def kernel(x, wavelet_indices, wavelet_values, inverse_wavelet_indices,
           inverse_wavelet_values, kernel, filt):
    h = _matmul(x, kernel)

    nblk1, colsp1, rowsp1, valsp1 = _prep_edges(
        inverse_wavelet_indices[0], inverse_wavelet_indices[1],
        inverse_wavelet_values)
    p1 = _make_spmm(nblk1)(h, colsp1, rowsp1, valsp1)
    h1 = _combine_filt(p1, filt.reshape(N, 1))

    nblk2, colsp2, rowsp2, valsp2 = _prep_edges(
        wavelet_indices[0], wavelet_indices[1], wavelet_values)
    p2 = _make_spmm(nblk2)(h1, colsp2, rowsp2, valsp2)
    return _combine(p2)


# speedup vs baseline: 3.6482x; 1.2007x over previous
import functools

import jax
import jax.numpy as jnp
from jax import lax
from jax.experimental import pallas as pl
from jax.experimental.pallas import tpu as pltpu
from jax.experimental.pallas import tpu_sc as plsc

N = 10000
D = 128
NT = 16
NW = 32
EB = 128
RPT = 640
RPL = N - 15 * RPT
RCH = 40
SBLK = 16
MB = 1000


def _mm_body(x_ref, k_ref, o_ref):
    o_ref[...] = jnp.dot(x_ref[...], k_ref[...],
                         preferred_element_type=jnp.float32)


_matmul = pl.pallas_call(
    _mm_body,
    grid=(N // MB,),
    in_specs=[
        pl.BlockSpec((MB, D), lambda i: (i, 0)),
        pl.BlockSpec((D, D), lambda i: (0, 0)),
    ],
    out_specs=pl.BlockSpec((MB, D), lambda i: (i, 0)),
    out_shape=jax.ShapeDtypeStruct((N, D), jnp.float32),
)


def _comb_filt_body(p_ref, f_ref, o_ref):
    o_ref[...] = (p_ref[0] + p_ref[1]) * f_ref[...]


_combine_filt = pl.pallas_call(
    _comb_filt_body,
    grid=(N // MB,),
    in_specs=[
        pl.BlockSpec((2, MB, D), lambda i: (0, i, 0)),
        pl.BlockSpec((MB, 1), lambda i: (i, 0)),
    ],
    out_specs=pl.BlockSpec((MB, D), lambda i: (i, 0)),
    out_shape=jax.ShapeDtypeStruct((N, D), jnp.float32),
)


def _comb_body(p_ref, o_ref):
    o_ref[...] = p_ref[0] + p_ref[1]


_combine = pl.pallas_call(
    _comb_body,
    grid=(N // MB,),
    in_specs=[pl.BlockSpec((2, MB, D), lambda i: (0, i, 0))],
    out_specs=pl.BlockSpec((MB, D), lambda i: (i, 0)),
    out_shape=jax.ShapeDtypeStruct((N, D), jnp.float32),
)


@functools.lru_cache(maxsize=None)
def _make_spmm(nblk: int):
    mesh = plsc.VectorSubcoreMesh(core_axis_name="c", subcore_axis_name="s")
    scratch = [
        pltpu.VMEM_SHARED((N, D), jnp.float32),
        pltpu.VMEM((SBLK, EB), jnp.int32),
        pltpu.VMEM((SBLK, EB), jnp.int32),
        pltpu.VMEM((SBLK, EB), jnp.float32),
        pltpu.VMEM((EB, D), jnp.float32),
        pltpu.VMEM((EB, D), jnp.float32),
        pltpu.VMEM((RCH, D), jnp.float32),
        pltpu.SemaphoreType.DMA,
        pltpu.SemaphoreType.DMA,
    ]

    def body(h, colsp, rowsp, valsp, out,
             acc, colsb, rowsb, valsb, gbuf_a, gbuf_b, obuf, sem_a, sem_b):
        c = lax.axis_index("c")
        s = lax.axis_index("s")
        base = s * RPT
        nrow = jnp.where(s == NT - 1, RPL, RPT)

        def zb(i, carry):
            for f in range(D // 16):
                obuf[i, pl.ds(f * 16, 16)] = jnp.zeros((16,), jnp.float32)
            return carry
        lax.fori_loop(0, RCH, zb, 0)

        def zc(k2, carry):
            pltpu.sync_copy(obuf, acc.at[pl.ds(base + k2 * RCH, RCH)])
            return carry
        lax.fori_loop(0, nrow // RCH, zc, 0)

        w = s * 2 + c
        plsc.subcore_barrier()

        def scale(buf, j):
            def se(g, c2):
                vv = valsb[j, pl.ds(g * 16, 16)]
                for u in range(16):
                    i = g * 16 + u
                    v = jnp.full((16,), vv[u], jnp.float32)
                    for f in range(D // 16):
                        sl = pl.ds(f * 16, 16)
                        buf[i, sl] = buf[i, sl] * v
                return c2
            lax.fori_loop(0, EB // 16, se, 0)

        def sb_loop(sb, carry):
            pltpu.sync_copy(colsp.at[w].at[pl.ds(sb * SBLK, SBLK)], colsb)
            pltpu.sync_copy(rowsp.at[w].at[pl.ds(sb * SBLK, SBLK)], rowsb)
            pltpu.sync_copy(valsp.at[w].at[pl.ds(sb * SBLK, SBLK)], valsb)

            pltpu.async_copy(h.at[colsb.at[0]], gbuf_a, sem_a)

            def blk2(j2, c1):
                ja = 2 * j2
                jb = 2 * j2 + 1
                pltpu.make_async_copy(h.at[colsb.at[ja]], gbuf_a,
                                      sem_a).wait()
                pltpu.async_copy(h.at[colsb.at[jb]], gbuf_b, sem_b)
                scale(gbuf_a, ja)
                pltpu.sync_copy(gbuf_a, acc.at[rowsb.at[ja]], add=True)

                pltpu.make_async_copy(h.at[colsb.at[jb]], gbuf_b,
                                      sem_b).wait()

                @pl.when(j2 < SBLK // 2 - 1)
                def _():
                    pltpu.async_copy(h.at[colsb.at[ja + 2]], gbuf_a, sem_a)

                scale(gbuf_b, jb)
                pltpu.sync_copy(gbuf_b, acc.at[rowsb.at[jb]], add=True)
                return c1
            lax.fori_loop(0, SBLK // 2, blk2, 0)
            return carry
        lax.fori_loop(0, nblk // SBLK, sb_loop, 0)

        plsc.subcore_barrier()

        def wb(k2, carry):
            pltpu.sync_copy(acc.at[pl.ds(base + k2 * RCH, RCH)], obuf)
            pltpu.sync_copy(obuf, out.at[c].at[pl.ds(base + k2 * RCH, RCH)])
            return carry
        lax.fori_loop(0, nrow // RCH, wb, 0)

    return pl.kernel(
        body,
        out_type=jax.ShapeDtypeStruct((2, N, D), jnp.float32),
        mesh=mesh,
        scratch_types=scratch,
    )


def _prep_edges(rows, cols, vals):
    e = rows.shape[0]
    nblk = -(-e // (NW * EB * SBLK)) * SBLK
    tot = NW * nblk * EB
    pad = tot - e
    rows_p = jnp.concatenate([rows, jnp.zeros((pad,), rows.dtype)])
    cols_p = jnp.concatenate([cols, jnp.zeros((pad,), cols.dtype)])
    vals_p = jnp.concatenate([vals, jnp.zeros((pad,), vals.dtype)])
    shape = (NW, nblk, EB)
    return (nblk, cols_p.reshape(shape), rows_p.reshape(shape),
            vals_p.reshape(shape))


def kernel(x, wavelet_indices, wavelet_values, inverse_wavelet_indices,
           inverse_wavelet_values, kernel, filt):
    h = _matmul(x, kernel)

    nblk1, colsp1, rowsp1, valsp1 = _prep_edges(
        inverse_wavelet_indices[0], inverse_wavelet_indices[1],
        inverse_wavelet_values)
    p1 = _make_spmm(nblk1)(h, colsp1, rowsp1, valsp1)
    h1 = _combine_filt(p1, filt.reshape(N, 1))

    nblk2, colsp2, rowsp2, valsp2 = _prep_edges(
        wavelet_indices[0], wavelet_indices[1], wavelet_values)
    p2 = _make_spmm(nblk2)(h1, colsp2, rowsp2, valsp2)
    return _combine(p2)
